# baseline (speedup 1.0000x reference)
import functools

import jax
import jax.numpy as jnp
from jax import lax
from jax.experimental import pallas as pl
from jax.experimental.pallas import tpu as pltpu
from jax.experimental.pallas import tpu_sc as plsc

N_NODES = 10000
D = 128
N_PAD = 10112
N_DEG = 10240
NW = 32
EC = 112
NCHUNK = 92
E_PAD = NW * NCHUNK * EC
NPT = N_PAD // 16
TRASH = N_NODES

NC0 = 92
NC1 = 92
assert NC0 + NC1 == 2 * NCHUNK and NC0 % 4 == 0 and NC1 % 4 == 0

_mesh = plsc.VectorSubcoreMesh(core_axis_name="c", subcore_axis_name="s")


@functools.partial(
    pl.kernel,
    mesh=_mesh,
    out_type=jax.ShapeDtypeStruct((2, N_DEG), jnp.float32),
    scratch_types=[
        pltpu.VMEM((EC,), jnp.int32),
        pltpu.VMEM((EC,), jnp.int32),
        pltpu.VMEM((EC,), jnp.int32),
        pltpu.VMEM((EC,), jnp.int32),
        pltpu.VMEM((EC,), jnp.float32),
        pltpu.VMEM((N_DEG // 16,), jnp.float32),
        pltpu.VMEM_SHARED((N_DEG,), jnp.float32),
        pltpu.SemaphoreType.DMA,
        pltpu.SemaphoreType.DMA,
    ],
)
def _deg_kernel(dsts_hbm, out_hbm, db0, db1, db2, db3,
                ones_v, zb_v, acc_sh, dsem, ssem):
    c = lax.axis_index("c")
    s = lax.axis_index("s")
    base = jnp.where(c == 0, s * NC0, 16 * NC0 + s * NC1)
    nch = jnp.where(c == 0, NC0, NC1)
    npt = N_DEG // 16
    dstb = (db0, db1, db2, db3)

    def fill_ones(i, carry):
        ones_v[pl.ds(i * 16, 16)] = jnp.ones((16,), jnp.float32)
        return carry

    lax.fori_loop(0, EC // 16, fill_ones, 0)

    def fill_zero(i, carry):
        zb_v[pl.ds(i * 16, 16)] = jnp.zeros((16,), jnp.float32)
        return carry

    lax.fori_loop(0, npt // 16, fill_zero, 0)

    pltpu.sync_copy(zb_v, acc_sh.at[pl.ds(s * npt, npt)])
    plsc.subcore_barrier()

    for t in range(4):
        pltpu.async_copy(dsts_hbm.at[base + t], dstb[t], dsem)

    def quad(q, carry):
        for t in range(4):
            cc = q * 4 + t
            db = dstb[t]
            pltpu.make_async_copy(dsts_hbm.at[base + cc], db, dsem).wait()
            pltpu.async_copy(ones_v, acc_sh.at[db], ssem, add=True)

            @pl.when(jnp.logical_and(cc >= 3, cc + 1 < nch))
            def _retire_and_refill():
                nb = dstb[(t + 1) % 4]
                pltpu.make_async_copy(ones_v, acc_sh.at[nb], ssem).wait()
                pltpu.async_copy(dsts_hbm.at[base + cc + 1], nb, dsem)

        return carry

    lax.fori_loop(0, nch // 4, quad, 0)

    for t in range(4):
        pltpu.make_async_copy(ones_v, acc_sh.at[dstb[t]], ssem).wait()

    plsc.subcore_barrier()
    pltpu.sync_copy(acc_sh.at[pl.ds(s * npt, npt)],
                    out_hbm.at[c, pl.ds(s * npt, npt)])


@functools.partial(
    pl.kernel,
    mesh=_mesh,
    out_type=jax.ShapeDtypeStruct((2, N_PAD, D), jnp.float32),
    scratch_types=[
        pltpu.VMEM((EC,), jnp.int32),
        pltpu.VMEM((EC,), jnp.int32),
        pltpu.VMEM((EC,), jnp.int32),
        pltpu.VMEM((EC,), jnp.int32),
        pltpu.VMEM((EC,), jnp.int32),
        pltpu.VMEM((EC,), jnp.int32),
        pltpu.VMEM((EC,), jnp.int32),
        pltpu.VMEM((EC,), jnp.int32),
        pltpu.VMEM((3, EC, D), jnp.float32),
        pltpu.VMEM_SHARED((N_PAD, D), jnp.float32),
        pltpu.SemaphoreType.DMA,
        pltpu.SemaphoreType.DMA,
        pltpu.SemaphoreType.DMA,
        pltpu.SemaphoreType.DMA,
    ],
)
def _agg_kernel(g_hbm, srcs_hbm, dsts_hbm, out_hbm,
                sb0, sb1, sb2, sb3, db0, db1, db2, db3,
                rows_v, acc_sh, isem, dsem, gsem, ssem):
    c = lax.axis_index("c")
    s = lax.axis_index("s")
    base = jnp.where(c == 0, s * NC0, 16 * NC0 + s * NC1)
    nch = jnp.where(c == 0, NC0, NC1)
    srcb = (sb0, sb1, sb2, sb3)
    dstb = (db0, db1, db2, db3)

    def zero_row(i, carry):
        def zero_lane(k, inner):
            rows_v[0, i, pl.ds(k * 16, 16)] = jnp.zeros((16,), jnp.float32)
            return inner

        lax.fori_loop(0, D // 16, zero_lane, 0)
        return carry

    lax.fori_loop(0, EC, zero_row, 0)

    for k in range(NPT // EC):
        pltpu.sync_copy(rows_v.at[0], acc_sh.at[pl.ds(s * NPT + k * EC, EC)])
    rem = NPT % EC
    if rem:
        pltpu.sync_copy(rows_v.at[0, pl.ds(0, rem)],
                        acc_sh.at[pl.ds(s * NPT + (NPT // EC) * EC, rem)])
    plsc.subcore_barrier()

    for t in range(4):
        pltpu.async_copy(srcs_hbm.at[base + t], srcb[t], isem)
        pltpu.async_copy(dsts_hbm.at[base + t], dstb[t], dsem)
    for t in range(2):
        pltpu.make_async_copy(srcs_hbm.at[base + t], srcb[t], isem).wait()
        pltpu.async_copy(g_hbm.at[srcb[t]], rows_v.at[t], gsem)

    def quad(q, carry):
        for t in range(4):
            cc = q * 4 + t
            sb, db = srcb[t], dstb[t]
            rb = rows_v.at[lax.rem(cc, 3)]
            pltpu.make_async_copy(dsts_hbm.at[base + cc], db, dsem).wait()
            pltpu.make_async_copy(g_hbm.at[sb], rb, gsem).wait()
            pltpu.async_copy(rb, acc_sh.at[db], ssem, add=True)

            @pl.when(cc >= 1)
            def _retire_prev_scatter():
                pltpu.make_async_copy(rows_v.at[0],
                                      acc_sh.at[dstb[(t + 3) % 4]],
                                      ssem).wait()

                @pl.when(cc + 3 < nch)
                def _refill_dst():
                    pltpu.async_copy(dsts_hbm.at[base + cc + 3],
                                     dstb[(t + 3) % 4], dsem)

            @pl.when(cc + 2 < nch)
            def _fire_gather():
                sbn = srcb[(t + 2) % 4]
                pltpu.make_async_copy(srcs_hbm.at[base + cc + 2],
                                      sbn, isem).wait()
                pltpu.async_copy(g_hbm.at[sbn],
                                 rows_v.at[lax.rem(cc + 2, 3)], gsem)

            @pl.when(cc + 4 < nch)
            def _refill_src():
                pltpu.async_copy(srcs_hbm.at[base + cc + 4], sb, isem)

        return carry

    lax.fori_loop(0, nch // 4, quad, 0)

    pltpu.make_async_copy(rows_v.at[0], acc_sh.at[dstb[3]], ssem).wait()

    plsc.subcore_barrier()
    pltpu.sync_copy(acc_sh.at[pl.ds(s * NPT, NPT)],
                    out_hbm.at[c, pl.ds(s * NPT, NPT)])


BLK = 512
_GRID = (N_PAD + BLK - 1) // BLK


def _tc1_body(x_ref, w_ref, d0_ref, d1_ref, g_ref, dinv_ref):
    deg = d0_ref[...] + d1_ref[...] + 1.0
    dinv = lax.rsqrt(jnp.maximum(deg, 1.0))
    sup = jnp.dot(x_ref[...], w_ref[...],
                  preferred_element_type=jnp.float32,
                  precision=lax.Precision.HIGHEST)
    g_ref[...] = sup * dinv
    dinv_ref[...] = dinv


_tc1 = pl.pallas_call(
    _tc1_body,
    grid=(_GRID,),
    in_specs=[
        pl.BlockSpec((BLK, D), lambda i: (i, 0)),
        pl.BlockSpec((D, D), lambda i: (0, 0)),
        pl.BlockSpec((BLK, 1), lambda i: (i, 0)),
        pl.BlockSpec((BLK, 1), lambda i: (i, 0)),
    ],
    out_specs=[
        pl.BlockSpec((BLK, D), lambda i: (i, 0)),
        pl.BlockSpec((BLK, 1), lambda i: (i, 0)),
    ],
    out_shape=[
        jax.ShapeDtypeStruct((N_PAD, D), jnp.float32),
        jax.ShapeDtypeStruct((N_PAD, 1), jnp.float32),
    ],
)


def _tc2_body(p0_ref, p1_ref, g1_ref, dinv_ref, w_ref, g2_ref):
    acc = p0_ref[...] + p1_ref[...] + g1_ref[...]
    h = jnp.maximum(acc * dinv_ref[...], 0.0)
    sup = jnp.dot(h, w_ref[...],
                  preferred_element_type=jnp.float32,
                  precision=lax.Precision.HIGHEST)
    g2_ref[...] = sup * dinv_ref[...]


_tc2 = pl.pallas_call(
    _tc2_body,
    grid=(_GRID,),
    in_specs=[
        pl.BlockSpec((BLK, D), lambda i: (i, 0)),
        pl.BlockSpec((BLK, D), lambda i: (i, 0)),
        pl.BlockSpec((BLK, D), lambda i: (i, 0)),
        pl.BlockSpec((BLK, 1), lambda i: (i, 0)),
        pl.BlockSpec((D, D), lambda i: (0, 0)),
    ],
    out_specs=pl.BlockSpec((BLK, D), lambda i: (i, 0)),
    out_shape=jax.ShapeDtypeStruct((N_PAD, D), jnp.float32),
)


def _tc3_body(p0_ref, p1_ref, g2_ref, dinv_ref, out_ref):
    acc = p0_ref[...] + p1_ref[...] + g2_ref[...]
    out_ref[...] = acc * dinv_ref[...]


_tc3 = pl.pallas_call(
    _tc3_body,
    grid=(_GRID,),
    in_specs=[
        pl.BlockSpec((BLK, D), lambda i: (i, 0)),
        pl.BlockSpec((BLK, D), lambda i: (i, 0)),
        pl.BlockSpec((BLK, D), lambda i: (i, 0)),
        pl.BlockSpec((BLK, 1), lambda i: (i, 0)),
    ],
    out_specs=pl.BlockSpec((BLK, D), lambda i: (i, 0)),
    out_shape=jax.ShapeDtypeStruct((N_PAD, D), jnp.float32),
)


def kernel(x, edge_index, W1, W2):
    src = edge_index[0].astype(jnp.int32)
    dst = edge_index[1].astype(jnp.int32)
    npad_e = E_PAD - src.shape[0]
    fill_src = jnp.full((npad_e,), TRASH, dtype=jnp.int32)
    fill_dst = TRASH + (jnp.arange(npad_e, dtype=jnp.int32) % (N_PAD - TRASH))
    srcs = jnp.concatenate([src, fill_src]).reshape(NW * NCHUNK, EC)
    dsts = jnp.concatenate([dst, fill_dst]).reshape(NW * NCHUNK, EC)
    x_p = jnp.zeros((N_PAD, D), jnp.float32).at[:N_NODES].set(x)

    degp = _deg_kernel(dsts)
    d0 = degp[0, :N_PAD].reshape(N_PAD, 1)
    d1 = degp[1, :N_PAD].reshape(N_PAD, 1)

    g1, dinv = _tc1(x_p, W1, d0, d1)
    p = _agg_kernel(g1, srcs, dsts)
    g2 = _tc2(p[0], p[1], g1, dinv, W2)
    p2 = _agg_kernel(g2, srcs, dsts)
    out = _tc3(p2[0], p2[1], g2, dinv)
    return out[:N_NODES]


# speedup vs baseline: 3.8452x; 3.8452x over previous
import functools

import jax
import jax.numpy as jnp
from jax import lax
from jax.experimental import pallas as pl
from jax.experimental.pallas import tpu as pltpu
from jax.experimental.pallas import tpu_sc as plsc

N_NODES = 10000
D = 128
N_PAD = 10112
N_DEG = 10240
NW = 32
EC = 128
NCHUNK = 80
E_PAD = NW * NCHUNK * EC
NPT = N_PAD // 16
TRASH = N_NODES

NC0 = 80
NC1 = 80
assert NC0 + NC1 == 2 * NCHUNK and NC0 % 4 == 0 and NC1 % 4 == 0

_mesh = plsc.VectorSubcoreMesh(core_axis_name="c", subcore_axis_name="s")


@functools.partial(
    pl.kernel,
    mesh=_mesh,
    out_type=jax.ShapeDtypeStruct((2, N_DEG), jnp.float32),
    scratch_types=[
        pltpu.VMEM((EC,), jnp.int32),
        pltpu.VMEM((EC,), jnp.int32),
        pltpu.VMEM((EC,), jnp.int32),
        pltpu.VMEM((EC,), jnp.int32),
        pltpu.VMEM((EC,), jnp.float32),
        pltpu.VMEM((N_DEG // 16,), jnp.float32),
        pltpu.VMEM_SHARED((N_DEG,), jnp.float32),
        pltpu.SemaphoreType.DMA,
        pltpu.SemaphoreType.DMA,
    ],
)
def _deg_kernel(dsts_hbm, out_hbm, db0, db1, db2, db3,
                ones_v, zb_v, acc_sh, dsem, ssem):
    c = lax.axis_index("c")
    s = lax.axis_index("s")
    base = jnp.where(c == 0, s * NC0, 16 * NC0 + s * NC1)
    nch = jnp.where(c == 0, NC0, NC1)
    npt = N_DEG // 16
    dstb = (db0, db1, db2, db3)

    def fill_ones(i, carry):
        ones_v[pl.ds(i * 16, 16)] = jnp.ones((16,), jnp.float32)
        return carry

    lax.fori_loop(0, EC // 16, fill_ones, 0)

    def fill_zero(i, carry):
        zb_v[pl.ds(i * 16, 16)] = jnp.zeros((16,), jnp.float32)
        return carry

    lax.fori_loop(0, npt // 16, fill_zero, 0)

    pltpu.sync_copy(zb_v, acc_sh.at[pl.ds(s * npt, npt)])
    plsc.subcore_barrier()

    for t in range(4):
        pltpu.async_copy(dsts_hbm.at[base + t], dstb[t], dsem)

    def quad(q, carry):
        for t in range(4):
            cc = q * 4 + t
            db = dstb[t]
            pltpu.make_async_copy(dsts_hbm.at[base + cc], db, dsem).wait()
            pltpu.async_copy(ones_v, acc_sh.at[db], ssem, add=True)

            @pl.when(jnp.logical_and(cc >= 3, cc + 1 < nch))
            def _retire_and_refill():
                nb = dstb[(t + 1) % 4]
                pltpu.make_async_copy(ones_v, acc_sh.at[nb], ssem).wait()
                pltpu.async_copy(dsts_hbm.at[base + cc + 1], nb, dsem)

        return carry

    lax.fori_loop(0, nch // 4, quad, 0)

    for t in range(4):
        pltpu.make_async_copy(ones_v, acc_sh.at[dstb[t]], ssem).wait()

    plsc.subcore_barrier()
    pltpu.sync_copy(acc_sh.at[pl.ds(s * npt, npt)],
                    out_hbm.at[c, pl.ds(s * npt, npt)])


@functools.partial(
    pl.kernel,
    mesh=_mesh,
    out_type=jax.ShapeDtypeStruct((2, N_PAD, D), jnp.float32),
    scratch_types=[
        pltpu.VMEM((EC,), jnp.int32),
        pltpu.VMEM((EC,), jnp.int32),
        pltpu.VMEM((EC,), jnp.int32),
        pltpu.VMEM((EC,), jnp.int32),
        pltpu.VMEM((EC,), jnp.int32),
        pltpu.VMEM((EC,), jnp.int32),
        pltpu.VMEM((EC,), jnp.int32),
        pltpu.VMEM((EC,), jnp.int32),
        pltpu.VMEM((EC, D), jnp.float32),
        pltpu.VMEM((EC, D), jnp.float32),
        pltpu.VMEM_SHARED((N_PAD, D), jnp.float32),
        pltpu.SemaphoreType.DMA,
        pltpu.SemaphoreType.DMA,
        pltpu.SemaphoreType.DMA,
    ],
)
def _agg_kernel(g_hbm, srcs_hbm, dsts_hbm, out_hbm,
                sb0, sb1, sb2, sb3, db0, db1, db2, db3,
                rows0, rows1, acc_sh, isem, dsem, gsem):
    c = lax.axis_index("c")
    s = lax.axis_index("s")
    base = jnp.where(c == 0, s * NC0, 16 * NC0 + s * NC1)
    nch = jnp.where(c == 0, NC0, NC1)
    srcb = (sb0, sb1, sb2, sb3)
    dstb = (db0, db1, db2, db3)
    rowsb = (rows0, rows1)

    def zero_row(i, carry):
        def zero_lane(k, inner):
            rows0[i, pl.ds(k * 16, 16)] = jnp.zeros((16,), jnp.float32)
            return inner

        lax.fori_loop(0, D // 16, zero_lane, 0)
        return carry

    lax.fori_loop(0, EC, zero_row, 0)

    for k in range(NPT // EC):
        pltpu.sync_copy(rows0, acc_sh.at[pl.ds(s * NPT + k * EC, EC)])
    rem = NPT % EC
    if rem:
        pltpu.sync_copy(rows0.at[pl.ds(0, rem)],
                        acc_sh.at[pl.ds(s * NPT + (NPT // EC) * EC, rem)])
    plsc.subcore_barrier()

    for t in range(4):
        pltpu.async_copy(srcs_hbm.at[base + t], srcb[t], isem)
        pltpu.async_copy(dsts_hbm.at[base + t], dstb[t], dsem)
    for t in range(2):
        pltpu.make_async_copy(srcs_hbm.at[base + t], srcb[t], isem).wait()
        pltpu.async_copy(g_hbm.at[srcb[t]], rowsb[t], gsem)

    def quad(q, carry):
        for t in range(4):
            cc = q * 4 + t
            sb, db, rb = srcb[t], dstb[t], rowsb[t % 2]
            pltpu.make_async_copy(dsts_hbm.at[base + cc], db, dsem).wait()
            pltpu.make_async_copy(g_hbm.at[sb], rb, gsem).wait()
            pltpu.sync_copy(rb, acc_sh.at[db], add=True)

            @pl.when(cc + 2 < nch)
            def _fire_gather():
                sbn = srcb[(t + 2) % 4]
                pltpu.make_async_copy(srcs_hbm.at[base + cc + 2],
                                      sbn, isem).wait()
                pltpu.async_copy(g_hbm.at[sbn], rb, gsem)

            @pl.when(cc + 4 < nch)
            def _refill_idx():
                pltpu.async_copy(srcs_hbm.at[base + cc + 4], sb, isem)
                pltpu.async_copy(dsts_hbm.at[base + cc + 4], db, dsem)

        return carry

    lax.fori_loop(0, nch // 4, quad, 0)
    plsc.subcore_barrier()
    pltpu.sync_copy(acc_sh.at[pl.ds(s * NPT, NPT)],
                    out_hbm.at[c, pl.ds(s * NPT, NPT)])


BLK = 512
_GRID = (N_PAD + BLK - 1) // BLK


def _tc1_body(x_ref, w_ref, d0_ref, d1_ref, g_ref, dinv_ref):
    deg = d0_ref[...] + d1_ref[...] + 1.0
    dinv = lax.rsqrt(jnp.maximum(deg, 1.0))
    sup = jnp.dot(x_ref[...], w_ref[...],
                  preferred_element_type=jnp.float32,
                  precision=lax.Precision.HIGHEST)
    g_ref[...] = sup * dinv
    dinv_ref[...] = dinv


_tc1 = pl.pallas_call(
    _tc1_body,
    grid=(_GRID,),
    in_specs=[
        pl.BlockSpec((BLK, D), lambda i: (i, 0)),
        pl.BlockSpec((D, D), lambda i: (0, 0)),
        pl.BlockSpec((BLK, 1), lambda i: (i, 0)),
        pl.BlockSpec((BLK, 1), lambda i: (i, 0)),
    ],
    out_specs=[
        pl.BlockSpec((BLK, D), lambda i: (i, 0)),
        pl.BlockSpec((BLK, 1), lambda i: (i, 0)),
    ],
    out_shape=[
        jax.ShapeDtypeStruct((N_PAD, D), jnp.float32),
        jax.ShapeDtypeStruct((N_PAD, 1), jnp.float32),
    ],
)


def _tc2_body(p0_ref, p1_ref, g1_ref, dinv_ref, w_ref, g2_ref):
    acc = p0_ref[...] + p1_ref[...] + g1_ref[...]
    h = jnp.maximum(acc * dinv_ref[...], 0.0)
    sup = jnp.dot(h, w_ref[...],
                  preferred_element_type=jnp.float32,
                  precision=lax.Precision.HIGHEST)
    g2_ref[...] = sup * dinv_ref[...]


_tc2 = pl.pallas_call(
    _tc2_body,
    grid=(_GRID,),
    in_specs=[
        pl.BlockSpec((BLK, D), lambda i: (i, 0)),
        pl.BlockSpec((BLK, D), lambda i: (i, 0)),
        pl.BlockSpec((BLK, D), lambda i: (i, 0)),
        pl.BlockSpec((BLK, 1), lambda i: (i, 0)),
        pl.BlockSpec((D, D), lambda i: (0, 0)),
    ],
    out_specs=pl.BlockSpec((BLK, D), lambda i: (i, 0)),
    out_shape=jax.ShapeDtypeStruct((N_PAD, D), jnp.float32),
)


def _tc3_body(p0_ref, p1_ref, g2_ref, dinv_ref, out_ref):
    acc = p0_ref[...] + p1_ref[...] + g2_ref[...]
    out_ref[...] = acc * dinv_ref[...]


_tc3 = pl.pallas_call(
    _tc3_body,
    grid=(_GRID,),
    in_specs=[
        pl.BlockSpec((BLK, D), lambda i: (i, 0)),
        pl.BlockSpec((BLK, D), lambda i: (i, 0)),
        pl.BlockSpec((BLK, D), lambda i: (i, 0)),
        pl.BlockSpec((BLK, 1), lambda i: (i, 0)),
    ],
    out_specs=pl.BlockSpec((BLK, D), lambda i: (i, 0)),
    out_shape=jax.ShapeDtypeStruct((N_PAD, D), jnp.float32),
)


def kernel(x, edge_index, W1, W2):
    src = edge_index[0].astype(jnp.int32)
    dst = edge_index[1].astype(jnp.int32)
    npad_e = E_PAD - src.shape[0]
    fill = TRASH + (jnp.arange(npad_e, dtype=jnp.int32) % (N_PAD - TRASH))
    srcs = jnp.concatenate([src, fill]).reshape(NW * NCHUNK, EC)
    dsts = jnp.concatenate([dst, fill]).reshape(NW * NCHUNK, EC)
    x_p = jnp.zeros((N_PAD, D), jnp.float32).at[:N_NODES].set(x)

    degp = _deg_kernel(dsts)
    d0 = degp[0, :N_PAD].reshape(N_PAD, 1)
    d1 = degp[1, :N_PAD].reshape(N_PAD, 1)

    g1, dinv = _tc1(x_p, W1, d0, d1)
    p = _agg_kernel(g1, srcs, dsts)
    g2 = _tc2(p[0], p[1], g1, dinv, W2)
    p2 = _agg_kernel(g2, srcs, dsts)
    out = _tc3(p2[0], p2[1], g2, dinv)
    return out[:N_NODES]


# speedup vs baseline: 3.9220x; 1.0200x over previous
import functools

import jax
import jax.numpy as jnp
from jax import lax
from jax.experimental import pallas as pl
from jax.experimental.pallas import tpu as pltpu
from jax.experimental.pallas import tpu_sc as plsc

N_NODES = 10000
D = 128
N_PAD = 10112
N_DEG = 10240
NW = 32
EC = 128
NCHUNK = 80
E_PAD = NW * NCHUNK * EC
NPT = N_PAD // 16
TRASH = N_NODES

NC0 = 80
NC1 = 80
assert NC0 + NC1 == 2 * NCHUNK and NC0 % 4 == 0 and NC1 % 4 == 0

_mesh = plsc.VectorSubcoreMesh(core_axis_name="c", subcore_axis_name="s")


@functools.partial(
    pl.kernel,
    mesh=_mesh,
    out_type=jax.ShapeDtypeStruct((2, N_DEG), jnp.float32),
    scratch_types=[
        pltpu.VMEM((EC,), jnp.int32),
        pltpu.VMEM((EC,), jnp.int32),
        pltpu.VMEM((EC,), jnp.int32),
        pltpu.VMEM((EC,), jnp.int32),
        pltpu.VMEM((EC,), jnp.float32),
        pltpu.VMEM((N_DEG // 16,), jnp.float32),
        pltpu.VMEM_SHARED((N_DEG,), jnp.float32),
        pltpu.SemaphoreType.DMA,
        pltpu.SemaphoreType.DMA,
    ],
)
def _deg_kernel(dsts_hbm, out_hbm, db0, db1, db2, db3,
                ones_v, zb_v, acc_sh, dsem, ssem):
    c = lax.axis_index("c")
    s = lax.axis_index("s")
    base = jnp.where(c == 0, s * NC0, 16 * NC0 + s * NC1)
    nch = jnp.where(c == 0, NC0, NC1)
    npt = N_DEG // 16
    dstb = (db0, db1, db2, db3)

    def fill_ones(i, carry):
        ones_v[pl.ds(i * 16, 16)] = jnp.ones((16,), jnp.float32)
        return carry

    lax.fori_loop(0, EC // 16, fill_ones, 0)

    def fill_zero(i, carry):
        zb_v[pl.ds(i * 16, 16)] = jnp.zeros((16,), jnp.float32)
        return carry

    lax.fori_loop(0, npt // 16, fill_zero, 0)

    pltpu.sync_copy(zb_v, acc_sh.at[pl.ds(s * npt, npt)])
    plsc.subcore_barrier()

    for t in range(4):
        pltpu.async_copy(dsts_hbm.at[base + t], dstb[t], dsem)

    def quad(q, carry):
        for t in range(4):
            cc = q * 4 + t
            db = dstb[t]
            pltpu.make_async_copy(dsts_hbm.at[base + cc], db, dsem).wait()
            pltpu.async_copy(ones_v, acc_sh.at[db], ssem, add=True)

            @pl.when(jnp.logical_and(cc >= 3, cc + 1 < nch))
            def _retire_and_refill():
                nb = dstb[(t + 1) % 4]
                pltpu.make_async_copy(ones_v, acc_sh.at[nb], ssem).wait()
                pltpu.async_copy(dsts_hbm.at[base + cc + 1], nb, dsem)

        return carry

    lax.fori_loop(0, nch // 4, quad, 0)

    for t in range(4):
        pltpu.make_async_copy(ones_v, acc_sh.at[dstb[t]], ssem).wait()

    plsc.subcore_barrier()
    pltpu.sync_copy(acc_sh.at[pl.ds(s * npt, npt)],
                    out_hbm.at[c, pl.ds(s * npt, npt)])


@functools.partial(
    pl.kernel,
    mesh=_mesh,
    out_type=jax.ShapeDtypeStruct((2, N_PAD, D), jnp.float32),
    scratch_types=[
        pltpu.VMEM((EC,), jnp.int32),
        pltpu.VMEM((EC,), jnp.int32),
        pltpu.VMEM((EC,), jnp.int32),
        pltpu.VMEM((EC,), jnp.int32),
        pltpu.VMEM((EC,), jnp.int32),
        pltpu.VMEM((EC,), jnp.int32),
        pltpu.VMEM((EC,), jnp.int32),
        pltpu.VMEM((EC,), jnp.int32),
        pltpu.VMEM((EC, D), jnp.float32),
        pltpu.VMEM((EC, D), jnp.float32),
        pltpu.VMEM_SHARED((N_PAD, D), jnp.float32),
        pltpu.SemaphoreType.DMA,
        pltpu.SemaphoreType.DMA,
        pltpu.SemaphoreType.DMA,
    ],
)
def _agg_kernel(g_hbm, srcs_hbm, dsts_hbm, out_hbm,
                sb0, sb1, sb2, sb3, db0, db1, db2, db3,
                rows0, rows1, acc_sh, isem, dsem, gsem):
    c = lax.axis_index("c")
    s = lax.axis_index("s")
    base = jnp.where(c == 0, s * NC0, 16 * NC0 + s * NC1)
    nch = jnp.where(c == 0, NC0, NC1)
    srcb = (sb0, sb1, sb2, sb3)
    dstb = (db0, db1, db2, db3)
    rowsb = (rows0, rows1)

    def zero_row(i, carry):
        def zero_lane(k, inner):
            rows0[i, pl.ds(k * 16, 16)] = jnp.zeros((16,), jnp.float32)
            return inner

        lax.fori_loop(0, D // 16, zero_lane, 0)
        return carry

    lax.fori_loop(0, EC, zero_row, 0)

    for k in range(NPT // EC):
        pltpu.sync_copy(rows0, acc_sh.at[pl.ds(s * NPT + k * EC, EC)])
    rem = NPT % EC
    if rem:
        pltpu.sync_copy(rows0.at[pl.ds(0, rem)],
                        acc_sh.at[pl.ds(s * NPT + (NPT // EC) * EC, rem)])
    plsc.subcore_barrier()

    for t in range(4):
        pltpu.async_copy(srcs_hbm.at[base + t], srcb[t], isem)
        pltpu.async_copy(dsts_hbm.at[base + t], dstb[t], dsem)
    for t in range(2):
        pltpu.make_async_copy(srcs_hbm.at[base + t], srcb[t], isem).wait()
        pltpu.async_copy(g_hbm.at[srcb[t]], rowsb[t], gsem)

    def quad(q, carry):
        for t in range(4):
            cc = q * 4 + t
            sb, db, rb = srcb[t], dstb[t], rowsb[t % 2]
            pltpu.make_async_copy(dsts_hbm.at[base + cc], db, dsem).wait()
            pltpu.make_async_copy(g_hbm.at[sb], rb, gsem).wait()
            pltpu.sync_copy(rb, acc_sh.at[db], add=True)

            @pl.when(cc + 2 < nch)
            def _fire_gather():
                sbn = srcb[(t + 2) % 4]
                pltpu.make_async_copy(srcs_hbm.at[base + cc + 2],
                                      sbn, isem).wait()
                pltpu.async_copy(g_hbm.at[sbn], rb, gsem)

            @pl.when(cc + 4 < nch)
            def _refill_idx():
                pltpu.async_copy(srcs_hbm.at[base + cc + 4], sb, isem)
                pltpu.async_copy(dsts_hbm.at[base + cc + 4], db, dsem)

        return carry

    lax.fori_loop(0, nch // 4, quad, 0)
    plsc.subcore_barrier()
    pltpu.sync_copy(acc_sh.at[pl.ds(s * NPT, NPT)],
                    out_hbm.at[c, pl.ds(s * NPT, NPT)])


BLK = 512
_GRID = (N_PAD + BLK - 1) // BLK


def _tc0_body(x_ref, w_ref, sup_ref):
    sup_ref[...] = jnp.dot(x_ref[...], w_ref[...],
                           preferred_element_type=jnp.float32,
                           precision=lax.Precision.HIGHEST)


_tc0 = pl.pallas_call(
    _tc0_body,
    grid=(_GRID,),
    in_specs=[
        pl.BlockSpec((BLK, D), lambda i: (i, 0)),
        pl.BlockSpec((D, D), lambda i: (0, 0)),
    ],
    out_specs=pl.BlockSpec((BLK, D), lambda i: (i, 0)),
    out_shape=jax.ShapeDtypeStruct((N_PAD, D), jnp.float32),
)


def _tc1_body(sup_ref, d0_ref, d1_ref, g_ref, dinv_ref):
    deg = d0_ref[...] + d1_ref[...] + 1.0
    dinv = lax.rsqrt(jnp.maximum(deg, 1.0))
    g_ref[...] = sup_ref[...] * dinv
    dinv_ref[...] = dinv


_tc1 = pl.pallas_call(
    _tc1_body,
    grid=(_GRID,),
    in_specs=[
        pl.BlockSpec((BLK, D), lambda i: (i, 0)),
        pl.BlockSpec((BLK, 1), lambda i: (i, 0)),
        pl.BlockSpec((BLK, 1), lambda i: (i, 0)),
    ],
    out_specs=[
        pl.BlockSpec((BLK, D), lambda i: (i, 0)),
        pl.BlockSpec((BLK, 1), lambda i: (i, 0)),
    ],
    out_shape=[
        jax.ShapeDtypeStruct((N_PAD, D), jnp.float32),
        jax.ShapeDtypeStruct((N_PAD, 1), jnp.float32),
    ],
)


def _tc2_body(p0_ref, p1_ref, g1_ref, dinv_ref, w_ref, g2_ref):
    acc = p0_ref[...] + p1_ref[...] + g1_ref[...]
    h = jnp.maximum(acc * dinv_ref[...], 0.0)
    sup = jnp.dot(h, w_ref[...],
                  preferred_element_type=jnp.float32,
                  precision=lax.Precision.HIGHEST)
    g2_ref[...] = sup * dinv_ref[...]


_tc2 = pl.pallas_call(
    _tc2_body,
    grid=(_GRID,),
    in_specs=[
        pl.BlockSpec((BLK, D), lambda i: (i, 0)),
        pl.BlockSpec((BLK, D), lambda i: (i, 0)),
        pl.BlockSpec((BLK, D), lambda i: (i, 0)),
        pl.BlockSpec((BLK, 1), lambda i: (i, 0)),
        pl.BlockSpec((D, D), lambda i: (0, 0)),
    ],
    out_specs=pl.BlockSpec((BLK, D), lambda i: (i, 0)),
    out_shape=jax.ShapeDtypeStruct((N_PAD, D), jnp.float32),
)


def _tc3_body(p0_ref, p1_ref, g2_ref, dinv_ref, out_ref):
    acc = p0_ref[...] + p1_ref[...] + g2_ref[...]
    out_ref[...] = acc * dinv_ref[...]


_tc3 = pl.pallas_call(
    _tc3_body,
    grid=(_GRID,),
    in_specs=[
        pl.BlockSpec((BLK, D), lambda i: (i, 0)),
        pl.BlockSpec((BLK, D), lambda i: (i, 0)),
        pl.BlockSpec((BLK, D), lambda i: (i, 0)),
        pl.BlockSpec((BLK, 1), lambda i: (i, 0)),
    ],
    out_specs=pl.BlockSpec((BLK, D), lambda i: (i, 0)),
    out_shape=jax.ShapeDtypeStruct((N_NODES, D), jnp.float32),
)


def kernel(x, edge_index, W1, W2):
    src = edge_index[0].astype(jnp.int32)
    dst = edge_index[1].astype(jnp.int32)
    npad_e = E_PAD - src.shape[0]
    fill = TRASH + (jnp.arange(npad_e, dtype=jnp.int32) % (N_PAD - TRASH))
    srcs = jnp.concatenate([src, fill]).reshape(NW * NCHUNK, EC)
    dsts = jnp.concatenate([dst, fill]).reshape(NW * NCHUNK, EC)

    degp = _deg_kernel(dsts)
    sup1 = _tc0(x, W1)
    d0 = degp[0, :N_PAD].reshape(N_PAD, 1)
    d1 = degp[1, :N_PAD].reshape(N_PAD, 1)

    g1, dinv = _tc1(sup1, d0, d1)
    p = _agg_kernel(g1, srcs, dsts)
    g2 = _tc2(p[0], p[1], g1, dinv, W2)
    p2 = _agg_kernel(g2, srcs, dsts)
    return _tc3(p2[0], p2[1], g2, dinv)


# speedup vs baseline: 4.0731x; 1.0385x over previous
import functools

import jax
import jax.numpy as jnp
from jax import lax
from jax.experimental import pallas as pl
from jax.experimental.pallas import tpu as pltpu
from jax.experimental.pallas import tpu_sc as plsc

N_NODES = 10000
D = 128
N_PAD = 10112
N_DEG = 10240
NW = 32
EC = 128
NCHUNK = 80
E_PAD = NW * NCHUNK * EC
NPT = N_PAD // 16
TRASH = N_NODES

NC0 = 80
NC1 = 80
assert NC0 + NC1 == 2 * NCHUNK and NC0 % 4 == 0 and NC1 % 4 == 0

_mesh = plsc.VectorSubcoreMesh(core_axis_name="c", subcore_axis_name="s")


@functools.partial(
    pl.kernel,
    mesh=_mesh,
    out_type=jax.ShapeDtypeStruct((2, N_DEG), jnp.float32),
    scratch_types=[
        pltpu.VMEM((EC,), jnp.int32),
        pltpu.VMEM((EC,), jnp.int32),
        pltpu.VMEM((EC,), jnp.int32),
        pltpu.VMEM((EC,), jnp.int32),
        pltpu.VMEM((EC,), jnp.float32),
        pltpu.VMEM((N_DEG // 16,), jnp.float32),
        pltpu.VMEM_SHARED((N_DEG,), jnp.float32),
        pltpu.SemaphoreType.DMA,
        pltpu.SemaphoreType.DMA,
    ],
)
def _deg_kernel(dsts_hbm, out_hbm, db0, db1, db2, db3,
                ones_v, zb_v, acc_sh, dsem, ssem):
    c = lax.axis_index("c")
    s = lax.axis_index("s")
    base = jnp.where(c == 0, s * NC0, 16 * NC0 + s * NC1)
    nch = jnp.where(c == 0, NC0, NC1)
    npt = N_DEG // 16
    dstb = (db0, db1, db2, db3)

    def fill_ones(i, carry):
        ones_v[pl.ds(i * 16, 16)] = jnp.ones((16,), jnp.float32)
        return carry

    lax.fori_loop(0, EC // 16, fill_ones, 0)

    def fill_zero(i, carry):
        zb_v[pl.ds(i * 16, 16)] = jnp.zeros((16,), jnp.float32)
        return carry

    lax.fori_loop(0, npt // 16, fill_zero, 0)

    pltpu.sync_copy(zb_v, acc_sh.at[pl.ds(s * npt, npt)])
    plsc.subcore_barrier()

    for t in range(4):
        pltpu.async_copy(dsts_hbm.at[base + t], dstb[t], dsem)

    def quad(q, carry):
        for t in range(4):
            cc = q * 4 + t
            db = dstb[t]
            pltpu.make_async_copy(dsts_hbm.at[base + cc], db, dsem).wait()
            pltpu.async_copy(ones_v, acc_sh.at[db], ssem, add=True)

            @pl.when(jnp.logical_and(cc >= 3, cc + 1 < nch))
            def _retire_and_refill():
                nb = dstb[(t + 1) % 4]
                pltpu.make_async_copy(ones_v, acc_sh.at[nb], ssem).wait()
                pltpu.async_copy(dsts_hbm.at[base + cc + 1], nb, dsem)

        return carry

    lax.fori_loop(0, nch // 4, quad, 0)

    for t in range(4):
        pltpu.make_async_copy(ones_v, acc_sh.at[dstb[t]], ssem).wait()

    plsc.subcore_barrier()
    pltpu.sync_copy(acc_sh.at[pl.ds(s * npt, npt)],
                    out_hbm.at[c, pl.ds(s * npt, npt)])


@functools.partial(
    pl.kernel,
    mesh=_mesh,
    out_type=[jax.ShapeDtypeStruct((N_PAD, D), jnp.float32),
              jax.ShapeDtypeStruct((N_PAD, D), jnp.float32)],
    scratch_types=[
        pltpu.VMEM((EC,), jnp.int32),
        pltpu.VMEM((EC,), jnp.int32),
        pltpu.VMEM((EC,), jnp.int32),
        pltpu.VMEM((EC,), jnp.int32),
        pltpu.VMEM((EC,), jnp.int32),
        pltpu.VMEM((EC,), jnp.int32),
        pltpu.VMEM((EC,), jnp.int32),
        pltpu.VMEM((EC,), jnp.int32),
        pltpu.VMEM((EC, D), jnp.float32),
        pltpu.VMEM((EC, D), jnp.float32),
        pltpu.VMEM_SHARED((N_PAD, D), jnp.float32),
        pltpu.SemaphoreType.DMA,
        pltpu.SemaphoreType.DMA,
        pltpu.SemaphoreType.DMA,
    ],
)
def _agg_kernel(g_hbm, srcs_hbm, dsts_hbm, out0_hbm, out1_hbm,
                sb0, sb1, sb2, sb3, db0, db1, db2, db3,
                rows0, rows1, acc_sh, isem, dsem, gsem):
    c = lax.axis_index("c")
    s = lax.axis_index("s")
    base = jnp.where(c == 0, s * NC0, 16 * NC0 + s * NC1)
    nch = jnp.where(c == 0, NC0, NC1)
    srcb = (sb0, sb1, sb2, sb3)
    dstb = (db0, db1, db2, db3)
    rowsb = (rows0, rows1)

    def zero_row(i, carry):
        def zero_lane(k, inner):
            rows0[i, pl.ds(k * 16, 16)] = jnp.zeros((16,), jnp.float32)
            return inner

        lax.fori_loop(0, D // 16, zero_lane, 0)
        return carry

    lax.fori_loop(0, EC, zero_row, 0)

    for k in range(NPT // EC):
        pltpu.sync_copy(rows0, acc_sh.at[pl.ds(s * NPT + k * EC, EC)])
    rem = NPT % EC
    if rem:
        pltpu.sync_copy(rows0.at[pl.ds(0, rem)],
                        acc_sh.at[pl.ds(s * NPT + (NPT // EC) * EC, rem)])
    plsc.subcore_barrier()

    for t in range(4):
        pltpu.async_copy(srcs_hbm.at[base + t], srcb[t], isem)
        pltpu.async_copy(dsts_hbm.at[base + t], dstb[t], dsem)
    for t in range(2):
        pltpu.make_async_copy(srcs_hbm.at[base + t], srcb[t], isem).wait()
        pltpu.async_copy(g_hbm.at[srcb[t]], rowsb[t], gsem)

    def quad(q, carry):
        for t in range(4):
            cc = q * 4 + t
            sb, db, rb = srcb[t], dstb[t], rowsb[t % 2]
            pltpu.make_async_copy(dsts_hbm.at[base + cc], db, dsem).wait()
            pltpu.make_async_copy(g_hbm.at[sb], rb, gsem).wait()
            pltpu.sync_copy(rb, acc_sh.at[db], add=True)

            @pl.when(cc + 2 < nch)
            def _fire_gather():
                sbn = srcb[(t + 2) % 4]
                pltpu.make_async_copy(srcs_hbm.at[base + cc + 2],
                                      sbn, isem).wait()
                pltpu.async_copy(g_hbm.at[sbn], rb, gsem)

            @pl.when(cc + 4 < nch)
            def _refill_idx():
                pltpu.async_copy(srcs_hbm.at[base + cc + 4], sb, isem)
                pltpu.async_copy(dsts_hbm.at[base + cc + 4], db, dsem)

        return carry

    lax.fori_loop(0, nch // 4, quad, 0)
    plsc.subcore_barrier()

    @pl.when(c == 0)
    def _wb0():
        pltpu.sync_copy(acc_sh.at[pl.ds(s * NPT, NPT)],
                        out0_hbm.at[pl.ds(s * NPT, NPT)])

    @pl.when(c == 1)
    def _wb1():
        pltpu.sync_copy(acc_sh.at[pl.ds(s * NPT, NPT)],
                        out1_hbm.at[pl.ds(s * NPT, NPT)])


BLK = 512
_GRID = (N_PAD + BLK - 1) // BLK


def _tc0_body(x_ref, w_ref, sup_ref):
    sup_ref[...] = jnp.dot(x_ref[...], w_ref[...],
                           preferred_element_type=jnp.float32,
                           precision=lax.Precision.HIGHEST)


_tc0 = pl.pallas_call(
    _tc0_body,
    grid=(_GRID,),
    in_specs=[
        pl.BlockSpec((BLK, D), lambda i: (i, 0)),
        pl.BlockSpec((D, D), lambda i: (0, 0)),
    ],
    out_specs=pl.BlockSpec((BLK, D), lambda i: (i, 0)),
    out_shape=jax.ShapeDtypeStruct((N_PAD, D), jnp.float32),
)


def _tc1_body(sup_ref, d0_ref, d1_ref, g_ref, dinv_ref):
    deg = d0_ref[...] + d1_ref[...] + 1.0
    dinv = lax.rsqrt(jnp.maximum(deg, 1.0))
    g_ref[...] = sup_ref[...] * dinv
    dinv_ref[...] = dinv


_tc1 = pl.pallas_call(
    _tc1_body,
    grid=(_GRID,),
    in_specs=[
        pl.BlockSpec((BLK, D), lambda i: (i, 0)),
        pl.BlockSpec((BLK, 1), lambda i: (i, 0)),
        pl.BlockSpec((BLK, 1), lambda i: (i, 0)),
    ],
    out_specs=[
        pl.BlockSpec((BLK, D), lambda i: (i, 0)),
        pl.BlockSpec((BLK, 1), lambda i: (i, 0)),
    ],
    out_shape=[
        jax.ShapeDtypeStruct((N_PAD, D), jnp.float32),
        jax.ShapeDtypeStruct((N_PAD, 1), jnp.float32),
    ],
)


def _tc2_body(p0_ref, p1_ref, g1_ref, dinv_ref, w_ref, g2_ref):
    acc = p0_ref[...] + p1_ref[...] + g1_ref[...]
    h = jnp.maximum(acc * dinv_ref[...], 0.0)
    sup = jnp.dot(h, w_ref[...],
                  preferred_element_type=jnp.float32,
                  precision=lax.Precision.HIGHEST)
    g2_ref[...] = sup * dinv_ref[...]


_tc2 = pl.pallas_call(
    _tc2_body,
    grid=(_GRID,),
    in_specs=[
        pl.BlockSpec((BLK, D), lambda i: (i, 0)),
        pl.BlockSpec((BLK, D), lambda i: (i, 0)),
        pl.BlockSpec((BLK, D), lambda i: (i, 0)),
        pl.BlockSpec((BLK, 1), lambda i: (i, 0)),
        pl.BlockSpec((D, D), lambda i: (0, 0)),
    ],
    out_specs=pl.BlockSpec((BLK, D), lambda i: (i, 0)),
    out_shape=jax.ShapeDtypeStruct((N_PAD, D), jnp.float32),
)


def _tc3_body(p0_ref, p1_ref, g2_ref, dinv_ref, out_ref):
    acc = p0_ref[...] + p1_ref[...] + g2_ref[...]
    out_ref[...] = acc * dinv_ref[...]


_tc3 = pl.pallas_call(
    _tc3_body,
    grid=(_GRID,),
    in_specs=[
        pl.BlockSpec((BLK, D), lambda i: (i, 0)),
        pl.BlockSpec((BLK, D), lambda i: (i, 0)),
        pl.BlockSpec((BLK, D), lambda i: (i, 0)),
        pl.BlockSpec((BLK, 1), lambda i: (i, 0)),
    ],
    out_specs=pl.BlockSpec((BLK, D), lambda i: (i, 0)),
    out_shape=jax.ShapeDtypeStruct((N_NODES, D), jnp.float32),
)


def kernel(x, edge_index, W1, W2):
    src = edge_index[0].astype(jnp.int32)
    dst = edge_index[1].astype(jnp.int32)
    npad_e = E_PAD - src.shape[0]
    fill = TRASH + (jnp.arange(npad_e, dtype=jnp.int32) % (N_PAD - TRASH))
    srcs = jnp.concatenate([src, fill]).reshape(NW * NCHUNK, EC)
    dsts = jnp.concatenate([dst, fill]).reshape(NW * NCHUNK, EC)

    degp = _deg_kernel(dsts)
    sup1 = _tc0(x, W1)
    d0 = degp[0, :N_PAD].reshape(N_PAD, 1)
    d1 = degp[1, :N_PAD].reshape(N_PAD, 1)

    g1, dinv = _tc1(sup1, d0, d1)
    p0, p1 = _agg_kernel(g1, srcs, dsts)
    g2 = _tc2(p0, p1, g1, dinv, W2)
    q0, q1 = _agg_kernel(g2, srcs, dsts)
    return _tc3(q0, q1, g2, dinv)
